# baseline (speedup 1.0000x reference)
import functools

import jax
import jax.numpy as jnp
from jax import lax
from jax.experimental import pallas as pl
from jax.experimental.pallas import tpu as pltpu
from jax.experimental.pallas import tpu_sc as plsc

DIM = 16
NUM_FEAT = 26
BATCH = 16384
TOTAL_ROWS = BATCH * NUM_FEAT
NC, NS, L = 2, 16, 16
NW = NC * NS
ROWS_PER_W = TOTAL_ROWS // NW
G = 128
NG = ROWS_PER_W // G


def _sc_embed(x2, off2, table):
    mesh = plsc.VectorSubcoreMesh(core_axis_name="c", subcore_axis_name="s")

    @functools.partial(
        pl.kernel,
        mesh=mesh,
        out_type=jax.ShapeDtypeStruct((TOTAL_ROWS, DIM), jnp.float32),
        compiler_params=pltpu.CompilerParams(use_tc_tiling_on_sc=False),
        scratch_types=[
            pltpu.VMEM((NG, G), jnp.int32),
            pltpu.VMEM((NG, G), jnp.int32),
            pltpu.VMEM((G, DIM), jnp.float32),
            pltpu.SemaphoreType.DMA,
        ],
    )
    def k(x2_hbm, off2_hbm, table_hbm, out_hbm, idx_v, off_v, rows_v, sem):
        wid = lax.axis_index("s") * NC + lax.axis_index("c")
        base = wid * ROWS_PER_W

        pltpu.sync_copy(x2_hbm.at[pl.ds(wid * NG, NG)], idx_v)
        pltpu.sync_copy(off2_hbm, off_v)

        def add_body(j, carry):
            for c in range(G // L):
                sl = pl.ds(c * L, L)
                idx_v[j, sl] = idx_v[j, sl] + off_v[j, sl]
            return carry

        lax.fori_loop(0, NG, add_body, 0)

        def gather_body(j, carry):
            pltpu.async_copy(table_hbm.at[idx_v.at[j]], rows_v, sem).wait()
            pltpu.sync_copy(rows_v, out_hbm.at[pl.ds(base + j * G, G)])
            return carry

        lax.fori_loop(0, NG, gather_body, 0)

    return k(x2, off2, table)


def kernel(x_cat, table, offsets):
    x2 = x_cat.astype(jnp.int32).reshape(TOTAL_ROWS // G, G)
    off2 = jnp.tile(offsets.astype(jnp.int32), ROWS_PER_W // NUM_FEAT)
    off2 = off2.reshape(NG, G)
    out = _sc_embed(x2, off2, table)
    return out.reshape(BATCH, NUM_FEAT, DIM)


# speedup vs baseline: 1.0749x; 1.0749x over previous
import functools

import jax
import jax.numpy as jnp
from jax import lax
from jax.experimental import pallas as pl
from jax.experimental.pallas import tpu as pltpu
from jax.experimental.pallas import tpu_sc as plsc

DIM = 16
NUM_FEAT = 26
BATCH = 16384
TOTAL_ROWS = BATCH * NUM_FEAT
NC, NS, L = 2, 16, 16
NW = NC * NS
ROWS_PER_W = TOTAL_ROWS // NW
G = 128
NG = ROWS_PER_W // G
CH = 13
NCH = NG // CH
CROWS = CH * G


def _sc_embed(x2, off2, table):
    mesh = plsc.VectorSubcoreMesh(core_axis_name="c", subcore_axis_name="s")

    @functools.partial(
        pl.kernel,
        mesh=mesh,
        out_type=jax.ShapeDtypeStruct((TOTAL_ROWS, DIM), jnp.float32),
        compiler_params=pltpu.CompilerParams(use_tc_tiling_on_sc=False),
        scratch_types=[
            pltpu.VMEM((NG, G), jnp.int32),
            pltpu.VMEM((CH, G), jnp.int32),
            pltpu.VMEM((2, CROWS, DIM), jnp.float32),
            pltpu.SemaphoreType.DMA,
            pltpu.SemaphoreType.DMA,
            pltpu.SemaphoreType.DMA,
            pltpu.SemaphoreType.DMA,
        ],
    )
    def k(x2_hbm, off2_hbm, table_hbm, out_hbm, idx_v, off_v, rows_v,
          gsem0, gsem1, wsem0, wsem1):
        gsems = (gsem0, gsem1)
        wsems = (wsem0, wsem1)
        wid = lax.axis_index("s") * NC + lax.axis_index("c")
        base = wid * ROWS_PER_W

        pltpu.sync_copy(x2_hbm.at[pl.ds(wid * NG, NG)], idx_v)
        pltpu.sync_copy(off2_hbm, off_v)

        def add_body(j, carry):
            for c in range(G // L):
                sl = pl.ds(c * L, L)
                idx_v[j, sl] = idx_v[j, sl] + off_v[j % CH, sl]
            return carry

        lax.fori_loop(0, NG, add_body, 0)

        def fire_gathers(g):
            b = g % 2
            return [
                pltpu.async_copy(
                    table_hbm.at[idx_v.at[g * CH + jj]],
                    rows_v.at[b, pl.ds(jj * G, G)],
                    gsems[b],
                )
                for jj in range(CH)
            ]

        pending_g = fire_gathers(0)
        pending_w = [None, None]
        for g in range(NCH):
            b = g % 2
            if g + 1 < NCH:
                if pending_w[(g + 1) % 2] is not None:
                    pending_w[(g + 1) % 2].wait()
                next_g = fire_gathers(g + 1)
            for h in pending_g:
                h.wait()
            if g + 1 < NCH:
                pending_g = next_g
            pending_w[b] = pltpu.async_copy(
                rows_v.at[b],
                out_hbm.at[pl.ds(base + g * CROWS, CROWS)],
                wsems[b],
            )
        pending_w[0].wait()
        pending_w[1].wait()

    return k(x2, off2, table)


def kernel(x_cat, table, offsets):
    x2 = x_cat.astype(jnp.int32).reshape(TOTAL_ROWS // G, G)
    off2 = jnp.tile(offsets.astype(jnp.int32), CROWS // NUM_FEAT)
    off2 = off2.reshape(CH, G)
    out = _sc_embed(x2, off2, table)
    return out.reshape(BATCH, NUM_FEAT, DIM)


# speedup vs baseline: 1.5659x; 1.4568x over previous
import functools

import jax
import jax.numpy as jnp
from jax import lax
from jax.experimental import pallas as pl
from jax.experimental.pallas import tpu as pltpu
from jax.experimental.pallas import tpu_sc as plsc

DIM = 16
NUM_FEAT = 26
BATCH = 16384
NC, NS, L = 2, 16, 16
NW = NC * NS
B_PER_W = BATCH // NW
G = 128
NQ = B_PER_W // G


def _sc_embed(xT, off2, table):
    mesh = plsc.VectorSubcoreMesh(core_axis_name="c", subcore_axis_name="s")

    @functools.partial(
        pl.kernel,
        mesh=mesh,
        out_type=jax.ShapeDtypeStruct((NUM_FEAT, BATCH, DIM), jnp.float32),
        compiler_params=pltpu.CompilerParams(use_tc_tiling_on_sc=False),
        scratch_types=[
            pltpu.VMEM((NUM_FEAT, B_PER_W), jnp.int32),
            pltpu.VMEM((NUM_FEAT, L), jnp.int32),
            pltpu.VMEM((2, NUM_FEAT, G, DIM), jnp.float32),
            pltpu.SemaphoreType.DMA,
            pltpu.SemaphoreType.DMA,
            pltpu.SemaphoreType.DMA,
            pltpu.SemaphoreType.DMA,
        ],
    )
    def k(xT_hbm, off2_hbm, table_hbm, out_hbm, idx_v, off_v, rows_v,
          gsem0, gsem1, wsem0, wsem1):
        gsems = (gsem0, gsem1)
        wsems = (wsem0, wsem1)
        wid = lax.axis_index("s") * NC + lax.axis_index("c")
        b0 = wid * B_PER_W

        pltpu.sync_copy(xT_hbm.at[:, pl.ds(b0, B_PER_W)], idx_v)
        pltpu.sync_copy(off2_hbm, off_v)

        def add_body(f, carry):
            off = off_v[f, :]
            for c in range(B_PER_W // L):
                sl = pl.ds(c * L, L)
                idx_v[f, sl] = idx_v[f, sl] + off
            return carry

        lax.fori_loop(0, NUM_FEAT, add_body, 0)

        def fire_gathers(q):
            b = q % 2
            return [
                pltpu.async_copy(
                    table_hbm.at[idx_v.at[f, pl.ds(q * G, G)]],
                    rows_v.at[b, f],
                    gsems[b],
                )
                for f in range(NUM_FEAT)
            ]

        pending_g = fire_gathers(0)
        pending_w = [None, None]
        for q in range(NQ):
            b = q % 2
            if q + 1 < NQ:
                if pending_w[(q + 1) % 2] is not None:
                    pending_w[(q + 1) % 2].wait()
                next_g = fire_gathers(q + 1)
            for h in pending_g:
                h.wait()
            if q + 1 < NQ:
                pending_g = next_g
            pending_w[b] = pltpu.async_copy(
                rows_v.at[b],
                out_hbm.at[:, pl.ds(b0 + q * G, G)],
                wsems[b],
            )
        pending_w[0].wait()
        pending_w[1].wait()

    return k(xT, off2, table)


def kernel(x_cat, table, offsets):
    xT = x_cat.astype(jnp.int32).T
    off2 = jnp.broadcast_to(offsets.astype(jnp.int32)[:, None], (NUM_FEAT, L))
    out3 = _sc_embed(xT, off2, table)
    return jnp.transpose(out3, (1, 0, 2))


# speedup vs baseline: 1.7153x; 1.0954x over previous
import functools

import jax
import jax.numpy as jnp
from jax import lax
from jax.experimental import pallas as pl
from jax.experimental.pallas import tpu as pltpu
from jax.experimental.pallas import tpu_sc as plsc

DIM = 16
NUM_FEAT = 26
BATCH = 16384
NC, NS, L = 2, 16, 16
NW = NC * NS
B_PER_W = BATCH // NW
G = 64
NQ = B_PER_W // G


def _sc_embed(xT, off2, table):
    mesh = plsc.VectorSubcoreMesh(core_axis_name="c", subcore_axis_name="s")

    @functools.partial(
        pl.kernel,
        mesh=mesh,
        out_type=jax.ShapeDtypeStruct((NUM_FEAT, DIM, BATCH), jnp.float32),
        compiler_params=pltpu.CompilerParams(
            use_tc_tiling_on_sc=False, needs_layout_passes=False),
        scratch_types=[
            pltpu.VMEM((NUM_FEAT, B_PER_W), jnp.int32),
            pltpu.VMEM((NUM_FEAT, L), jnp.int32),
            pltpu.VMEM((2, NUM_FEAT, G, DIM), jnp.float32),
            pltpu.VMEM((2, NUM_FEAT, DIM, G), jnp.float32),
            pltpu.SemaphoreType.DMA,
            pltpu.SemaphoreType.DMA,
            pltpu.SemaphoreType.DMA,
            pltpu.SemaphoreType.DMA,
        ],
    )
    def k(xT_hbm, off2_hbm, table_hbm, out_hbm, idx_v, off_v, rows_v, trows_v,
          gsem0, gsem1, wsem0, wsem1):
        gsems = (gsem0, gsem1)
        wsems = (wsem0, wsem1)
        wid = lax.axis_index("s") * NC + lax.axis_index("c")
        b0 = wid * B_PER_W

        pltpu.sync_copy(xT_hbm.at[:, pl.ds(b0, B_PER_W)], idx_v)
        pltpu.sync_copy(off2_hbm, off_v)

        def add_body(f, carry):
            off = off_v[f, :]
            for c in range(B_PER_W // L):
                sl = pl.ds(c * L, L)
                idx_v[f, sl] = idx_v[f, sl] + off
            return carry

        lax.fori_loop(0, NUM_FEAT, add_body, 0)

        iota = lax.iota(jnp.int32, L)

        def fire_gathers(q):
            b = q % 2
            return [
                pltpu.async_copy(
                    table_hbm.at[idx_v.at[f, pl.ds(q * G, G)]],
                    rows_v.at[b, f],
                    gsems[b],
                )
                for f in range(NUM_FEAT)
            ]

        def transpose_chunk(b):
            def body(f, carry):
                rows_f = rows_v.at[b, f]
                for d in range(DIM):
                    idx1 = jnp.full((L,), d, jnp.int32)
                    for c in range(G // L):
                        v = plsc.load_gather(rows_f, [iota + (c * L), idx1])
                        trows_v[b, f, d, pl.ds(c * L, L)] = v
                return carry
            lax.fori_loop(0, NUM_FEAT, body, 0)

        pending_g = fire_gathers(0)
        pending_w = [None, None]
        for q in range(NQ):
            b = q % 2
            if q + 1 < NQ:
                next_g = fire_gathers(q + 1)
            for h in pending_g:
                h.wait()
            if pending_w[b] is not None:
                pending_w[b].wait()
            transpose_chunk(b)
            if q + 1 < NQ:
                pending_g = next_g
            pending_w[b] = pltpu.async_copy(
                trows_v.at[b],
                out_hbm.at[:, :, pl.ds(b0 + q * G, G)],
                wsems[b],
            )
        pending_w[0].wait()
        pending_w[1].wait()

    return k(xT, off2, table)


def kernel(x_cat, table, offsets):
    xT = x_cat.astype(jnp.int32).T
    off2 = jnp.broadcast_to(offsets.astype(jnp.int32)[:, None], (NUM_FEAT, L))
    out3 = _sc_embed(xT, off2, table)
    return jnp.transpose(out3, (2, 0, 1))
